# baseline (speedup 1.0000x reference)
import functools

import jax
import jax.numpy as jnp
from jax import lax
from jax.experimental import pallas as pl
from jax.experimental.pallas import tpu as pltpu
from jax.experimental.pallas import tpu_sc as plsc

_T, _BCOL = 200, 4096
_V, _D = 1_000_000, 32

_info = plsc.get_sparse_core_info()
_NC, _NS = _info.num_cores, _info.num_subcores
_NW = _NC * _NS
_NTILES = (_T // 8) * (_BCOL // 128)
_TPW = _NTILES // _NW

_mesh = plsc.VectorSubcoreMesh(core_axis_name="c", subcore_axis_name="s")


@functools.partial(
    pl.kernel,
    mesh=_mesh,
    out_type=jax.ShapeDtypeStruct((_T, 4, _BCOL // 128, 8, 128), jnp.float32),
    scratch_types=[
        pltpu.VMEM((1024,), jnp.int32),
        pltpu.VMEM((1024, _D), jnp.float32),
        pltpu.VMEM((4, 8, 128), jnp.float32),
        pltpu.SemaphoreType.DMA,
    ],
    compiler_params=pltpu.CompilerParams(
        use_tc_tiling_on_sc=False, needs_layout_passes=False
    ),
)
def _emb_lookup(idx_hbm, table_hbm, out_hbm, idx_v, rows_v, obuf, gsem):
    wid = lax.axis_index("s") * _NC + lax.axis_index("c")
    base = wid * _TPW
    iota16 = lax.iota(jnp.int32, 16)

    def jbody(j, carry):
        k = base + j
        R = k >> 5
        C = k & 31
        pltpu.sync_copy(idx_hbm.at[k], idx_v)
        pltpu.async_copy(table_hbm.at[idx_v], rows_v, gsem).wait()

        def trbody(tr, c2):
            rbase = tr * 128 + iota16
            for Rd in range(4):
                for dr in range(8):
                    cvec = jnp.full((16,), 8 * Rd + dr, jnp.int32)
                    for b0 in range(0, 128, 16):
                        v = plsc.load_gather(rows_v, [rbase + b0, cvec])
                        obuf[Rd, dr, pl.ds(b0, 16)] = v
            t = 8 * R + tr
            pltpu.sync_copy(obuf, out_hbm.at[t, :, C])
            return c2

        lax.fori_loop(0, 8, trbody, 0)
        return carry

    lax.fori_loop(0, _TPW, jbody, 0)


def kernel(inputs, embedding_weight):
    idx4 = (
        inputs.astype(jnp.int32)
        .reshape(_T // 8, 8, _BCOL // 128, 128)
        .transpose(0, 2, 1, 3)
        .reshape(_NTILES, 1024)
    )
    out5 = _emb_lookup(idx4, embedding_weight)
    return out5.transpose(0, 2, 4, 1, 3).reshape(_T, _BCOL, _D)


# speedup vs baseline: 1.0919x; 1.0919x over previous
import functools

import jax
import jax.numpy as jnp
from jax import lax
from jax.experimental import pallas as pl
from jax.experimental.pallas import tpu as pltpu
from jax.experimental.pallas import tpu_sc as plsc

_T, _BCOL = 200, 4096
_V, _D = 1_000_000, 32

_info = plsc.get_sparse_core_info()
_NC, _NS = _info.num_cores, _info.num_subcores
_NW = _NC * _NS
_NTILES = (_T // 8) * (_BCOL // 128)
_TPW = _NTILES // _NW
_NHALF = 2 * _TPW

_mesh = plsc.VectorSubcoreMesh(core_axis_name="c", subcore_axis_name="s")


@functools.partial(
    pl.kernel,
    mesh=_mesh,
    out_type=jax.ShapeDtypeStruct((_T, 4, _BCOL // 128, 8, 128), jnp.float32),
    scratch_types=[
        pltpu.VMEM((2, 512), jnp.int32),
        pltpu.VMEM((2, 512, _D), jnp.float32),
        pltpu.VMEM((2, 4, 4, 8, 128), jnp.float32),
        pltpu.SemaphoreType.DMA,
        pltpu.SemaphoreType.DMA,
        pltpu.SemaphoreType.DMA,
    ],
    compiler_params=pltpu.CompilerParams(
        use_tc_tiling_on_sc=False, needs_layout_passes=False
    ),
)
def _emb_lookup(idx_hbm, table_hbm, out_hbm, idx_v, rows_v, obuf, isem, gsem, osem):
    wid = lax.axis_index("s") * _NC + lax.axis_index("c")
    base = wid * _TPW
    iota16 = lax.iota(jnp.int32, 16)

    def issue_idx(k, h, p):
        pltpu.async_copy(idx_hbm.at[k, pl.ds(h * 512, 512)], idx_v.at[p], isem)

    def wait_idx(p):
        pltpu.make_async_copy(idx_hbm.at[0, pl.ds(0, 512)], idx_v.at[p], isem).wait()

    def issue_gather(p):
        pltpu.async_copy(table_hbm.at[idx_v.at[p]], rows_v.at[p], gsem)

    def wait_gather(p):
        pltpu.make_async_copy(
            table_hbm.at[pl.ds(0, 512)], rows_v.at[p], gsem
        ).wait()

    def drain_out(p):
        pltpu.make_async_copy(
            out_hbm.at[pl.ds(0, 4), :, 0], obuf.at[p], osem
        ).wait()

    def transpose_and_store(p, R, C, h):
        rows = rows_v.at[p]
        ob = obuf.at[p]

        def qbody(q, carry):
            rq = q * 128
            for Rd in range(4):
                for dr in range(8):
                    cvec = jnp.full((16,), 8 * Rd + dr, jnp.int32)
                    for b0 in range(0, 128, 16):
                        v = plsc.load_gather(rows, [rq + b0 + iota16, cvec])
                        ob[q, Rd, dr, pl.ds(b0, 16)] = v
            return carry

        lax.fori_loop(0, 4, qbody, 0)
        pltpu.async_copy(ob, out_hbm.at[pl.ds(8 * R + 4 * h, 4), :, C], osem)

    pltpu.sync_copy(idx_hbm.at[base, pl.ds(0, 512)], idx_v.at[0])
    issue_gather(0)
    issue_idx(base, 1, 1)

    def body(i2, carry):
        k = base + i2
        R = k >> 5
        C = k & 31
        kn = k + 1
        @pl.when(i2 > 0)
        def _():
            drain_out(0)

        wait_gather(0)
        wait_idx(1)
        issue_gather(1)

        @pl.when(i2 < _TPW - 1)
        def _():
            issue_idx(kn, 0, 0)

        transpose_and_store(0, R, C, 0)

        @pl.when(i2 > 0)
        def _():
            drain_out(1)

        wait_gather(1)

        @pl.when(i2 < _TPW - 1)
        def _():
            wait_idx(0)
            issue_gather(0)
            issue_idx(kn, 1, 1)

        transpose_and_store(1, R, C, 1)
        return carry

    lax.fori_loop(0, _TPW, body, 0)
    drain_out(0)
    drain_out(1)


def kernel(inputs, embedding_weight):
    idx4 = (
        inputs.astype(jnp.int32)
        .reshape(_T // 8, 8, _BCOL // 128, 128)
        .transpose(0, 2, 1, 3)
        .reshape(_NTILES, 1024)
    )
    out5 = _emb_lookup(idx4, embedding_weight)
    return out5.transpose(0, 2, 4, 1, 3).reshape(_T, _BCOL, _D)


# speedup vs baseline: 1.2394x; 1.1351x over previous
import functools

import jax
import jax.numpy as jnp
from jax import lax
from jax.experimental import pallas as pl
from jax.experimental.pallas import tpu as pltpu
from jax.experimental.pallas import tpu_sc as plsc

_T, _BCOL = 200, 4096
_V, _D = 1_000_000, 32

_info = plsc.get_sparse_core_info()
_NC, _NS = _info.num_cores, _info.num_subcores
_NW = _NC * _NS
_NTILES = (_T // 8) * (_BCOL // 128)
_TPW = _NTILES // _NW
_OUT_WORDS = _T * _BCOL * _D

_mesh = plsc.VectorSubcoreMesh(core_axis_name="c", subcore_axis_name="s")


@functools.partial(
    pl.kernel,
    mesh=_mesh,
    out_type=jax.ShapeDtypeStruct((_OUT_WORDS,), jnp.float32),
    scratch_types=[
        pltpu.VMEM((2, 512), jnp.int32),
        pltpu.VMEM((2, 512, _D), jnp.float32),
        pltpu.VMEM((2, 16384), jnp.float32),
        pltpu.SemaphoreType.DMA,
        pltpu.SemaphoreType.DMA,
        pltpu.SemaphoreType.DMA,
    ],
    compiler_params=pltpu.CompilerParams(
        use_tc_tiling_on_sc=False, needs_layout_passes=False
    ),
)
def _emb_lookup(idx_hbm, table_hbm, out_hbm, idx_v, rows_v, obuf, isem, gsem, osem):
    wid = lax.axis_index("s") * _NC + lax.axis_index("c")
    base = wid * _TPW
    iota16 = lax.iota(jnp.int32, 16)
    c128iota = iota16 * 128
    c2048 = jnp.full((16,), 2048, jnp.int32)

    def issue_idx(k, h, p):
        pltpu.async_copy(idx_hbm.at[k, pl.ds(h * 512, 512)], idx_v.at[p], isem)

    def wait_idx(p):
        pltpu.make_async_copy(idx_hbm.at[0, pl.ds(0, 512)], idx_v.at[p], isem).wait()

    def issue_gather(p):
        pltpu.async_copy(table_hbm.at[idx_v.at[p]], rows_v.at[p], gsem)

    def wait_gather(p):
        pltpu.make_async_copy(
            table_hbm.at[pl.ds(0, 512)], rows_v.at[p], gsem
        ).wait()

    def drain_out(p):
        pltpu.make_async_copy(
            out_hbm.at[pl.ds(0, 16384)], obuf.at[p], osem
        ).wait()

    def transpose_and_store(p, R, C, h):
        rows = rows_v.at[p]
        ob = obuf.at[p]

        def rbody(i, carry):
            for u in range(8):
                r = i * 8 + u
                s = (r >> 7) * 4096 + (r & 127)
                ivec0 = s + c128iota
                ivec1 = ivec0 + c2048
                v0 = rows[r, pl.ds(0, 16)]
                v1 = rows[r, pl.ds(16, 16)]
                plsc.store_scatter(ob, [ivec0], v0)
                plsc.store_scatter(ob, [ivec1], v1)
            return carry

        lax.fori_loop(0, 64, rbody, 0)
        for q in range(4):
            for Rd in range(4):
                dst = (8 * R + 4 * h + q) * 131072 + Rd * 32768 + C * 1024
                pltpu.async_copy(
                    ob.at[pl.ds((q * 4 + Rd) * 1024, 1024)],
                    out_hbm.at[pl.ds(dst, 1024)],
                    osem,
                )

    pltpu.sync_copy(idx_hbm.at[base, pl.ds(0, 512)], idx_v.at[0])
    issue_gather(0)
    issue_idx(base, 1, 1)

    def body(i2, carry):
        k = base + i2
        R = k >> 5
        C = k & 31
        kn = k + 1
        @pl.when(i2 > 0)
        def _():
            drain_out(0)

        wait_gather(0)
        wait_idx(1)
        issue_gather(1)

        @pl.when(i2 < _TPW - 1)
        def _():
            issue_idx(kn, 0, 0)

        transpose_and_store(0, R, C, 0)

        @pl.when(i2 > 0)
        def _():
            drain_out(1)

        wait_gather(1)

        @pl.when(i2 < _TPW - 1)
        def _():
            wait_idx(0)
            issue_gather(0)
            issue_idx(kn, 1, 1)

        transpose_and_store(1, R, C, 1)
        return carry

    lax.fori_loop(0, _TPW, body, 0)
    drain_out(0)
    drain_out(1)


def kernel(inputs, embedding_weight):
    idx4 = (
        inputs.astype(jnp.int32)
        .reshape(_T // 8, 8, _BCOL // 128, 128)
        .transpose(0, 2, 1, 3)
        .reshape(_NTILES, 1024)
    )
    out_flat = _emb_lookup(idx4, embedding_weight)
    return (
        out_flat.reshape(_T, 4, _BCOL // 128, 8, 128)
        .transpose(0, 2, 4, 1, 3)
        .reshape(_T, _BCOL, _D)
    )


# speedup vs baseline: 1.6394x; 1.3227x over previous
import functools

import jax
import jax.numpy as jnp
from jax import lax
from jax.experimental import pallas as pl
from jax.experimental.pallas import tpu as pltpu
from jax.experimental.pallas import tpu_sc as plsc

_T, _BCOL = 200, 4096
_V, _D = 1_000_000, 32

_info = plsc.get_sparse_core_info()
_NC, _NS = _info.num_cores, _info.num_subcores
_NW = _NC * _NS
_NTILES = (_T // 8) * (_BCOL // 128)
_TPW = _NTILES // _NW
_OUT_WORDS = _T * _BCOL * _D

_mesh = plsc.VectorSubcoreMesh(core_axis_name="c", subcore_axis_name="s")


@functools.partial(
    pl.kernel,
    mesh=_mesh,
    out_type=jax.ShapeDtypeStruct((_OUT_WORDS,), jnp.float32),
    scratch_types=[
        pltpu.VMEM((2, 512), jnp.int32),
        pltpu.VMEM((2, 512, _D), jnp.float32),
        pltpu.VMEM((2, 16384), jnp.float32),
        pltpu.SemaphoreType.DMA,
        pltpu.SemaphoreType.DMA,
        pltpu.SemaphoreType.DMA,
    ],
    compiler_params=pltpu.CompilerParams(
        use_tc_tiling_on_sc=False, needs_layout_passes=False
    ),
)
def _emb_lookup(idx_hbm, table_hbm, out_hbm, idx_v, rows_v, obuf, isem, gsem, osem):
    wid = lax.axis_index("s") * _NC + lax.axis_index("c")
    base = wid * _TPW
    iota16 = lax.iota(jnp.int32, 16)
    c128iota = iota16 * 128
    c2048 = jnp.full((16,), 2048, jnp.int32)

    def issue_idx(k, h, p):
        pltpu.async_copy(idx_hbm.at[k, pl.ds(h * 512, 512)], idx_v.at[p], isem)

    def wait_idx(p):
        pltpu.make_async_copy(idx_hbm.at[0, pl.ds(0, 512)], idx_v.at[p], isem).wait()

    def issue_gather(p):
        pltpu.async_copy(table_hbm.at[idx_v.at[p]], rows_v.at[p], gsem)

    def wait_gather(p):
        pltpu.make_async_copy(
            table_hbm.at[pl.ds(0, 512)], rows_v.at[p], gsem
        ).wait()

    def drain_out(p):
        pltpu.make_async_copy(
            out_hbm.at[pl.ds(0, 16384)], obuf.at[p], osem
        ).wait()

    def transpose_and_store(p, R, C, h):
        rows = rows_v.at[p]
        ob = obuf.at[p]
        cvec0 = iota16
        cvec1 = iota16 + 16

        def qbody(q, carry):
            qb = q * 128
            ob_q = q * 4096
            for tc0 in range(128):
                tcv = (tc0 + iota16) & 127
                rvec = qb + tcv
                for dh, cvec in ((0, cvec0), (16, cvec1)):
                    v = plsc.load_gather(rows, [rvec, cvec])
                    plsc.store_scatter(
                        ob, [ob_q + dh * 128 + c128iota + tcv], v
                    )
            return carry

        lax.fori_loop(0, 4, qbody, 0)
        for q in range(4):
            for Rd in range(4):
                dst = (8 * R + 4 * h + q) * 131072 + Rd * 32768 + C * 1024
                pltpu.async_copy(
                    ob.at[pl.ds((q * 4 + Rd) * 1024, 1024)],
                    out_hbm.at[pl.ds(dst, 1024)],
                    osem,
                )

    pltpu.sync_copy(idx_hbm.at[base, pl.ds(0, 512)], idx_v.at[0])
    issue_gather(0)
    issue_idx(base, 1, 1)

    def body(i2, carry):
        k = base + i2
        R = k >> 5
        C = k & 31
        kn = k + 1
        @pl.when(i2 > 0)
        def _():
            drain_out(0)

        wait_gather(0)
        wait_idx(1)
        issue_gather(1)

        @pl.when(i2 < _TPW - 1)
        def _():
            issue_idx(kn, 0, 0)

        transpose_and_store(0, R, C, 0)

        @pl.when(i2 > 0)
        def _():
            drain_out(1)

        wait_gather(1)

        @pl.when(i2 < _TPW - 1)
        def _():
            wait_idx(0)
            issue_gather(0)
            issue_idx(kn, 1, 1)

        transpose_and_store(1, R, C, 1)
        return carry

    lax.fori_loop(0, _TPW, body, 0)
    drain_out(0)
    drain_out(1)


def kernel(inputs, embedding_weight):
    idx4 = (
        inputs.astype(jnp.int32)
        .reshape(_T // 8, 8, _BCOL // 128, 128)
        .transpose(0, 2, 1, 3)
        .reshape(_NTILES, 1024)
    )
    out_flat = _emb_lookup(idx4, embedding_weight)
    return (
        out_flat.reshape(_T, 4, _BCOL // 128, 8, 128)
        .transpose(0, 2, 4, 1, 3)
        .reshape(_T, _BCOL, _D)
    )


# speedup vs baseline: 1.6419x; 1.0015x over previous
import functools

import jax
import jax.numpy as jnp
from jax import lax
from jax.experimental import pallas as pl
from jax.experimental.pallas import tpu as pltpu
from jax.experimental.pallas import tpu_sc as plsc

_T, _BCOL = 200, 4096
_V, _D = 1_000_000, 32

_info = plsc.get_sparse_core_info()
_NC, _NS = _info.num_cores, _info.num_subcores
_NW = _NC * _NS
_NTILES = (_T // 8) * (_BCOL // 128)
_TPW = _NTILES // _NW
_OUT_WORDS = _T * _BCOL * _D

_mesh = plsc.VectorSubcoreMesh(core_axis_name="c", subcore_axis_name="s")


@functools.partial(
    pl.kernel,
    mesh=_mesh,
    out_type=jax.ShapeDtypeStruct((_OUT_WORDS,), jnp.float32),
    scratch_types=[
        pltpu.VMEM((2, 512), jnp.int32),
        pltpu.VMEM((2, 512, _D), jnp.float32),
        pltpu.VMEM((2, 16384), jnp.float32),
        pltpu.SemaphoreType.DMA,
        pltpu.SemaphoreType.DMA,
        pltpu.SemaphoreType.DMA,
    ],
    compiler_params=pltpu.CompilerParams(
        use_tc_tiling_on_sc=False, needs_layout_passes=False
    ),
)
def _emb_lookup(idx_hbm, table_hbm, out_hbm, idx_v, rows_v, obuf, isem, gsem, osem):
    wid = lax.axis_index("s") * _NC + lax.axis_index("c")
    base = wid * _TPW
    iota16 = lax.iota(jnp.int32, 16)
    c128iota = iota16 * 128
    c2048 = jnp.full((16,), 2048, jnp.int32)

    def issue_idx(k, h, p):
        pltpu.async_copy(idx_hbm.at[k, pl.ds(h * 512, 512)], idx_v.at[p], isem)

    def wait_idx(p):
        pltpu.make_async_copy(idx_hbm.at[0, pl.ds(0, 512)], idx_v.at[p], isem).wait()

    def issue_gather(p):
        pltpu.async_copy(table_hbm.at[idx_v.at[p]], rows_v.at[p], gsem)

    def wait_gather(p):
        pltpu.make_async_copy(
            table_hbm.at[pl.ds(0, 512)], rows_v.at[p], gsem
        ).wait()

    def drain_out(p):
        pltpu.make_async_copy(
            out_hbm.at[pl.ds(0, 16384)], obuf.at[p], osem
        ).wait()

    def transpose_and_store(p, R, C, h):
        rows = rows_v.at[p]
        ob = obuf.at[p]
        cvec0 = iota16
        cvec1 = iota16 + 16

        def qbody(q, carry):
            qb = q * 128
            ob_q = q * 4096
            for tc0 in range(128):
                tcv = tc0 + iota16
                if tc0 > 112:
                    tcv = tcv & 127
                rvec = qb + tcv
                ovec = c128iota + tcv
                for dh, cvec in ((0, cvec0), (16, cvec1)):
                    v = plsc.load_gather(rows, [rvec, cvec])
                    plsc.store_scatter(ob, [ob_q + dh * 128 + ovec], v)
            return carry

        lax.fori_loop(0, 4, qbody, 0)
        for q in range(4):
            for Rd in range(4):
                dst = (8 * R + 4 * h + q) * 131072 + Rd * 32768 + C * 1024
                pltpu.async_copy(
                    ob.at[pl.ds((q * 4 + Rd) * 1024, 1024)],
                    out_hbm.at[pl.ds(dst, 1024)],
                    osem,
                )

    pltpu.sync_copy(idx_hbm.at[base, pl.ds(0, 512)], idx_v.at[0])
    issue_gather(0)
    issue_idx(base, 1, 1)

    def body(i2, carry):
        k = base + i2
        R = k >> 5
        C = k & 31
        kn = k + 1
        @pl.when(i2 > 0)
        def _():
            drain_out(0)

        wait_gather(0)
        wait_idx(1)
        issue_gather(1)

        @pl.when(i2 < _TPW - 1)
        def _():
            issue_idx(kn, 0, 0)

        transpose_and_store(0, R, C, 0)

        @pl.when(i2 > 0)
        def _():
            drain_out(1)

        wait_gather(1)

        @pl.when(i2 < _TPW - 1)
        def _():
            wait_idx(0)
            issue_gather(0)
            issue_idx(kn, 1, 1)

        transpose_and_store(1, R, C, 1)
        return carry

    lax.fori_loop(0, _TPW, body, 0)
    drain_out(0)
    drain_out(1)


def kernel(inputs, embedding_weight):
    idx4 = (
        inputs.astype(jnp.int32)
        .reshape(_T // 8, 8, _BCOL // 128, 128)
        .transpose(0, 2, 1, 3)
        .reshape(_NTILES, 1024)
    )
    out_flat = _emb_lookup(idx4, embedding_weight)
    return (
        out_flat.reshape(_T, 4, _BCOL // 128, 8, 128)
        .transpose(0, 2, 4, 1, 3)
        .reshape(_T, _BCOL, _D)
    )
